# baseline (speedup 1.0000x reference)
import functools

import jax
import jax.numpy as jnp
from jax import lax
from jax.experimental import pallas as pl
from jax.experimental.pallas import tpu as pltpu


def _cmpex_pair(av, bv, aw, bw, s, base_bit_scalar, lane_desc):
    d = av - bv
    if lane_desc is not None:
        ex = (lane_desc * d) > 0
    elif base_bit_scalar is not None:
        ex = (base_bit_scalar * d) > 0
    else:
        ex = d > 0
    return (jnp.where(ex, bv, av), jnp.where(ex, av, bv),
            jnp.where(ex, bw, aw), jnp.where(ex, aw, bw))


def _wass_body(x_ref, y_ref, xw_ref, yw_ref, out_ref, vbuf, wbuf, *, rw):
    RT = vbuf.shape[0]
    C = vbuf.shape[1]
    HRT = RT // 2
    rb = RT.bit_length() - 1
    logm = (RT * C).bit_length() - 1
    rwb = rw.bit_length() - 1
    nw = RT // rw
    f32 = jnp.float32
    i32 = jnp.int32

    def fill_tot(b, acc):
        ax, ay = acc
        ax = ax + jnp.sum(xw_ref[pl.ds(b * rw, rw), :])
        ay = ay + jnp.sum(yw_ref[pl.ds(b * rw, rw), :])
        vbuf[pl.ds(b * rw, rw), :] = x_ref[pl.ds(b * rw, rw), :]
        vbuf[pl.ds(HRT + b * rw, rw), :] = y_ref[pl.ds(b * rw, rw), :]
        return ax, ay

    wx_tot, wy_tot = lax.fori_loop(0, nw // 2, fill_tot,
                                   (jnp.float32(0.0), jnp.float32(0.0)))
    inv_x = 1.0 / wx_tot
    inv_y = -1.0 / wy_tot

    def fill_w(b, _):
        wbuf[pl.ds(b * rw, rw), :] = xw_ref[pl.ds(b * rw, rw), :] * inv_x
        wbuf[pl.ds(HRT + b * rw, rw), :] = yw_ref[pl.ds(b * rw, rw), :] * inv_y
        return 0

    lax.fori_loop(0, nw // 2, fill_w, 0)

    riota = lax.broadcasted_iota(i32, (rw, C), 0)
    ciota = lax.broadcasted_iota(i32, (rw, C), 1)

    def desc_mult(s, base):
        if s < rb:
            dsc = (base >> s) & 1
            return (1 - 2 * dsc).astype(f32)
        return (1 - 2 * ((ciota >> (s - rb)) & 1)).astype(f32)

    def substage_inwin(vv, ww, t, s, base):
        j = 1 << t
        if j >= 8:
            nseg = rw // (2 * j)
            segs_v, segs_w = [], []
            for g in range(nseg):
                lo = g * 2 * j
                av, bv = vv[lo:lo + j, :], vv[lo + j:lo + 2 * j, :]
                aw, bw = ww[lo:lo + j, :], ww[lo + j:lo + 2 * j, :]
                if s < rwb:
                    if ((lo >> s) & 1) == 0:
                        na, nb, nwa, nwb = _cmpex_pair(av, bv, aw, bw, s, None, None)
                    else:
                        nb, na, nwb, nwa = _cmpex_pair(bv, av, bw, aw, s, None, None)
                elif s < rb:
                    dscf = desc_mult(s, base)
                    na, nb, nwa, nwb = _cmpex_pair(av, bv, aw, bw, s, dscf, None)
                else:
                    lmask = desc_mult(s, 0)[:j, :]
                    na, nb, nwa, nwb = _cmpex_pair(av, bv, aw, bw, s, None, lmask)
                segs_v += [na, nb]
                segs_w += [nwa, nwb]
            return (jnp.concatenate(segs_v, axis=0),
                    jnp.concatenate(segs_w, axis=0))
        hb = (riota >> t) & 1
        if s < rwb:
            db = (riota >> s) & 1
            sgn = ((1 - 2 * hb) * (1 - 2 * db)).astype(f32)
        elif s < rb:
            sgn = (1 - 2 * hb).astype(f32) * desc_mult(s, base)
        else:
            sgn = (1 - 2 * hb).astype(f32) * desc_mult(s, 0)
        hbf = hb.astype(f32)
        pv = hbf * pltpu.roll(vv, j, 0) + (1.0 - hbf) * pltpu.roll(vv, rw - j, 0)
        pw = hbf * pltpu.roll(ww, j, 0) + (1.0 - hbf) * pltpu.roll(ww, rw - j, 0)
        ex = sgn * (vv - pv) > 0
        return jnp.where(ex, pv, vv), jnp.where(ex, pw, ww)

    def window_pass(stages):
        def body(wi, _):
            base = wi * rw
            vv = vbuf[pl.ds(base, rw), :]
            ww = wbuf[pl.ds(base, rw), :]
            for (t, s) in stages:
                vv, ww = substage_inwin(vv, ww, t, s, base)
            vbuf[pl.ds(base, rw), :] = vv
            wbuf[pl.ds(base, rw), :] = ww
            return 0
        lax.fori_loop(0, nw, body, 0)

    def pair_pass(t, s):
        jw = 1 << (t - rwb)

        def body(i, _):
            b = (i // jw) * 2 * jw + (i % jw)
            lo = b * rw
            hi = lo + jw * rw
            av = vbuf[pl.ds(lo, rw), :]
            aw = wbuf[pl.ds(lo, rw), :]
            bv = vbuf[pl.ds(hi, rw), :]
            bw = wbuf[pl.ds(hi, rw), :]
            if s < rb:
                dscf = desc_mult(s, lo)
                na, nb, nwa, nwb = _cmpex_pair(av, bv, aw, bw, s, dscf, None)
            else:
                na, nb, nwa, nwb = _cmpex_pair(av, bv, aw, bw, s, None,
                                               desc_mult(s, 0))
            vbuf[pl.ds(lo, rw), :] = na
            wbuf[pl.ds(lo, rw), :] = nwa
            vbuf[pl.ds(hi, rw), :] = nb
            wbuf[pl.ds(hi, rw), :] = nwb
            return 0

        lax.fori_loop(0, nw // 2, body, 0)

    def lane_pass(t, s):
        jl = 1 << (t - rb)
        hb = (ciota >> (t - rb)) & 1
        db = (ciota >> (s - rb)) & 1
        sgn = ((1 - 2 * hb) * (1 - 2 * db)).astype(f32)
        hbf = hb.astype(f32)

        def body(wi, _):
            base = wi * rw
            vv = vbuf[pl.ds(base, rw), :]
            ww = wbuf[pl.ds(base, rw), :]
            pv = hbf * pltpu.roll(vv, jl, 1) + (1.0 - hbf) * pltpu.roll(vv, C - jl, 1)
            pw = hbf * pltpu.roll(ww, jl, 1) + (1.0 - hbf) * pltpu.roll(ww, C - jl, 1)
            ex = sgn * (vv - pv) > 0
            vbuf[pl.ds(base, rw), :] = jnp.where(ex, pv, vv)
            wbuf[pl.ds(base, rw), :] = jnp.where(ex, pw, ww)
            return 0

        lax.fori_loop(0, nw, body, 0)

    group_a = [(t, s) for s in range(1, rwb + 1) for t in range(s - 1, -1, -1)]
    window_pass(group_a)
    for s in range(rwb + 1, logm + 1):
        for t in range(s - 1, rwb - 1, -1):
            if t >= rb:
                lane_pass(t, s)
            else:
                pair_pass(t, s)
        window_pass([(t, s) for t in range(rwb - 1, -1, -1)])

    def local_scan(ww):
        n = 1
        while n < rw:
            ww = ww + jnp.concatenate(
                [jnp.zeros((n, C), f32), ww[: rw - n, :]], axis=0)
            n *= 2
        return ww

    def pass1(b, carry):
        ww = wbuf[pl.ds(b * rw, rw), :]
        incl = local_scan(ww)
        return carry + incl[rw - 1: rw, :]

    col_tot = lax.fori_loop(0, nw, pass1, jnp.zeros((1, C), f32))

    lane_incl = col_tot
    n = 1
    while n < C:
        lane_incl = lane_incl + jnp.concatenate(
            [jnp.zeros((1, n), f32), lane_incl[:, : C - n]], axis=1)
        n *= 2
    col_off = lane_incl - col_tot

    def pass2(b, carry):
        off, loss = carry
        ww = wbuf[pl.ds(b * rw, rw), :]
        vv = vbuf[pl.ds(b * rw, rw), :]
        incl_local = local_scan(ww)
        s_incl = incl_local + off
        s_excl = s_incl - ww
        loss = loss + jnp.sum((jnp.abs(s_excl) - jnp.abs(s_incl)) * vv)
        return off + incl_local[rw - 1: rw, :], loss

    _, loss = lax.fori_loop(0, nw, pass2, (col_off, jnp.float32(0.0)))
    out_ref[...] = loss.reshape(1, 1)


@functools.partial(jax.jit, static_argnames=("rows", "cols", "rw", "interpret"))
def _wass_loss(x, y, xw, yw, rows, cols, rw=512, interpret=False):
    body = functools.partial(_wass_body, rw=rw)
    f = pl.pallas_call(
        body,
        out_shape=jax.ShapeDtypeStruct((1, 1), jnp.float32),
        scratch_shapes=[
            pltpu.VMEM((2 * rows, cols), jnp.float32),
            pltpu.VMEM((2 * rows, cols), jnp.float32),
        ],
        interpret=interpret,
    )
    return f(
        x.reshape(rows, cols),
        y.reshape(rows, cols),
        xw.reshape(rows, cols),
        yw.reshape(rows, cols),
    )[0, 0]


def kernel(x, y, x_weights, y_weights, pre_sorted):
    del pre_sorted
    n = x.shape[0]
    cols = 128
    rows = n // cols
    return _wass_loss(x, y, x_weights, y_weights, rows, cols, rw=128)


# speedup vs baseline: 1.1409x; 1.1409x over previous
import functools

import jax
import jax.numpy as jnp
from jax import lax
from jax.experimental import pallas as pl
from jax.experimental.pallas import tpu as pltpu


def _cmpex_pair(av, bv, aw, bw, s, base_bit_scalar, lane_desc):
    d = av - bv
    if lane_desc is not None:
        ex = (lane_desc * d) > 0
    elif base_bit_scalar is not None:
        ex = (base_bit_scalar * d) > 0
    else:
        ex = d > 0
    return (jnp.where(ex, bv, av), jnp.where(ex, av, bv),
            jnp.where(ex, bw, aw), jnp.where(ex, aw, bw))


def _wass_body(x_ref, y_ref, xw_ref, yw_ref, out_ref, vbuf, wbuf, *, rw):
    RT = vbuf.shape[0]
    C = vbuf.shape[1]
    HRT = RT // 2
    rb = RT.bit_length() - 1
    logm = (RT * C).bit_length() - 1
    rwb = rw.bit_length() - 1
    nw = RT // rw
    f32 = jnp.float32
    i32 = jnp.int32

    def fill_tot(b, acc):
        ax, ay = acc
        ax = ax + jnp.sum(xw_ref[pl.ds(b * rw, rw), :])
        ay = ay + jnp.sum(yw_ref[pl.ds(b * rw, rw), :])
        vbuf[pl.ds(b * rw, rw), :] = x_ref[pl.ds(b * rw, rw), :]
        vbuf[pl.ds(HRT + b * rw, rw), :] = y_ref[pl.ds(b * rw, rw), :]
        return ax, ay

    wx_tot, wy_tot = lax.fori_loop(0, nw // 2, fill_tot,
                                   (jnp.float32(0.0), jnp.float32(0.0)))
    inv_x = 1.0 / wx_tot
    inv_y = -1.0 / wy_tot

    def fill_w(b, _):
        wbuf[pl.ds(b * rw, rw), :] = xw_ref[pl.ds(b * rw, rw), :] * inv_x
        wbuf[pl.ds(HRT + b * rw, rw), :] = yw_ref[pl.ds(b * rw, rw), :] * inv_y
        return 0

    lax.fori_loop(0, nw // 2, fill_w, 0)

    riota = lax.broadcasted_iota(i32, (rw, C), 0)
    ciota = lax.broadcasted_iota(i32, (rw, C), 1)

    def desc_mult(s, base):
        if s < rb:
            dsc = (base >> s) & 1
            return (1 - 2 * dsc).astype(f32)
        return (1 - 2 * ((ciota >> (s - rb)) & 1)).astype(f32)

    def substage_inwin(vv, ww, t, s, base):
        j = 1 << t
        if j >= 8:
            nseg = rw // (2 * j)
            segs_v, segs_w = [], []
            for g in range(nseg):
                lo = g * 2 * j
                av, bv = vv[lo:lo + j, :], vv[lo + j:lo + 2 * j, :]
                aw, bw = ww[lo:lo + j, :], ww[lo + j:lo + 2 * j, :]
                if s < rwb:
                    if ((lo >> s) & 1) == 0:
                        na, nb, nwa, nwb = _cmpex_pair(av, bv, aw, bw, s, None, None)
                    else:
                        nb, na, nwb, nwa = _cmpex_pair(bv, av, bw, aw, s, None, None)
                elif s < rb:
                    dscf = desc_mult(s, base)
                    na, nb, nwa, nwb = _cmpex_pair(av, bv, aw, bw, s, dscf, None)
                else:
                    lmask = desc_mult(s, 0)[:j, :]
                    na, nb, nwa, nwb = _cmpex_pair(av, bv, aw, bw, s, None, lmask)
                segs_v += [na, nb]
                segs_w += [nwa, nwb]
            return (jnp.concatenate(segs_v, axis=0),
                    jnp.concatenate(segs_w, axis=0))
        hb = (riota >> t) & 1
        if s < rwb:
            db = (riota >> s) & 1
            sgn = ((1 - 2 * hb) * (1 - 2 * db)).astype(f32)
        elif s < rb:
            sgn = (1 - 2 * hb).astype(f32) * desc_mult(s, base)
        else:
            sgn = (1 - 2 * hb).astype(f32) * desc_mult(s, 0)
        hbf = hb.astype(f32)
        pv = hbf * pltpu.roll(vv, j, 0) + (1.0 - hbf) * pltpu.roll(vv, rw - j, 0)
        pw = hbf * pltpu.roll(ww, j, 0) + (1.0 - hbf) * pltpu.roll(ww, rw - j, 0)
        ex = sgn * (vv - pv) > 0
        return jnp.where(ex, pv, vv), jnp.where(ex, pw, ww)

    def window_pass(stages):
        def body(wi, _):
            base = wi * rw
            vv = vbuf[pl.ds(base, rw), :]
            ww = wbuf[pl.ds(base, rw), :]
            for (t, s) in stages:
                vv, ww = substage_inwin(vv, ww, t, s, base)
            vbuf[pl.ds(base, rw), :] = vv
            wbuf[pl.ds(base, rw), :] = ww
            return 0
        lax.fori_loop(0, nw, body, 0)

    def pair_pass(t, s):
        jw = 1 << (t - rwb)

        def body(i, _):
            b = (i // jw) * 2 * jw + (i % jw)
            lo = b * rw
            hi = lo + jw * rw
            av = vbuf[pl.ds(lo, rw), :]
            aw = wbuf[pl.ds(lo, rw), :]
            bv = vbuf[pl.ds(hi, rw), :]
            bw = wbuf[pl.ds(hi, rw), :]
            if s < rb:
                dscf = desc_mult(s, lo)
                na, nb, nwa, nwb = _cmpex_pair(av, bv, aw, bw, s, dscf, None)
            else:
                na, nb, nwa, nwb = _cmpex_pair(av, bv, aw, bw, s, None,
                                               desc_mult(s, 0))
            vbuf[pl.ds(lo, rw), :] = na
            wbuf[pl.ds(lo, rw), :] = nwa
            vbuf[pl.ds(hi, rw), :] = nb
            wbuf[pl.ds(hi, rw), :] = nwb
            return 0

        lax.fori_loop(0, nw // 2, body, 0)

    def lane_substage(vv, ww, t, s):
        jl = 1 << (t - rb)
        hb = (ciota >> (t - rb)) & 1
        db = (ciota >> (s - rb)) & 1
        sgn = ((1 - 2 * hb) * (1 - 2 * db)).astype(f32)
        hbf = hb.astype(f32)
        pv = hbf * pltpu.roll(vv, jl, 1) + (1.0 - hbf) * pltpu.roll(vv, C - jl, 1)
        pw = hbf * pltpu.roll(ww, jl, 1) + (1.0 - hbf) * pltpu.roll(ww, C - jl, 1)
        ex = sgn * (vv - pv) > 0
        return jnp.where(ex, pv, vv), jnp.where(ex, pw, ww)

    def lane_run_pass(s, tlist):
        def body(wi, _):
            base = wi * rw
            vv = vbuf[pl.ds(base, rw), :]
            ww = wbuf[pl.ds(base, rw), :]
            for t in tlist:
                vv, ww = lane_substage(vv, ww, t, s)
            vbuf[pl.ds(base, rw), :] = vv
            wbuf[pl.ds(base, rw), :] = ww
            return 0

        lax.fori_loop(0, nw, body, 0)

    def pair_window_pass(s):
        def body(i, _):
            lo = (2 * i) * rw
            hi = lo + rw
            av = vbuf[pl.ds(lo, rw), :]
            aw = wbuf[pl.ds(lo, rw), :]
            bv = vbuf[pl.ds(hi, rw), :]
            bw = wbuf[pl.ds(hi, rw), :]
            if s < rb:
                na, nb, nwa, nwb = _cmpex_pair(av, bv, aw, bw, s,
                                               desc_mult(s, lo), None)
            else:
                na, nb, nwa, nwb = _cmpex_pair(av, bv, aw, bw, s, None,
                                               desc_mult(s, 0))
            for t in range(rwb - 1, -1, -1):
                na, nwa = substage_inwin(na, nwa, t, s, lo)
                nb, nwb = substage_inwin(nb, nwb, t, s, hi)
            vbuf[pl.ds(lo, rw), :] = na
            wbuf[pl.ds(lo, rw), :] = nwa
            vbuf[pl.ds(hi, rw), :] = nb
            wbuf[pl.ds(hi, rw), :] = nwb
            return 0

        lax.fori_loop(0, nw // 2, body, 0)

    group_a = [(t, s) for s in range(1, rwb + 1) for t in range(s - 1, -1, -1)]
    window_pass(group_a)
    for s in range(rwb + 1, logm + 1):
        lane_ts = list(range(s - 1, rb - 1, -1)) if s - 1 >= rb else []
        if lane_ts:
            lane_run_pass(s, lane_ts)
        for t in range(min(s - 1, rb - 1), rwb, -1):
            pair_pass(t, s)
        pair_window_pass(s)

    def local_scan(ww):
        n = 1
        while n < rw:
            ww = ww + jnp.concatenate(
                [jnp.zeros((n, C), f32), ww[: rw - n, :]], axis=0)
            n *= 2
        return ww

    def pass1(b, carry):
        ww = wbuf[pl.ds(b * rw, rw), :]
        incl = local_scan(ww)
        return carry + incl[rw - 1: rw, :]

    col_tot = lax.fori_loop(0, nw, pass1, jnp.zeros((1, C), f32))

    lane_incl = col_tot
    n = 1
    while n < C:
        lane_incl = lane_incl + jnp.concatenate(
            [jnp.zeros((1, n), f32), lane_incl[:, : C - n]], axis=1)
        n *= 2
    col_off = lane_incl - col_tot

    def pass2(b, carry):
        off, loss = carry
        ww = wbuf[pl.ds(b * rw, rw), :]
        vv = vbuf[pl.ds(b * rw, rw), :]
        incl_local = local_scan(ww)
        s_incl = incl_local + off
        s_excl = s_incl - ww
        loss = loss + jnp.sum((jnp.abs(s_excl) - jnp.abs(s_incl)) * vv)
        return off + incl_local[rw - 1: rw, :], loss

    _, loss = lax.fori_loop(0, nw, pass2, (col_off, jnp.float32(0.0)))
    out_ref[...] = loss.reshape(1, 1)


@functools.partial(jax.jit, static_argnames=("rows", "cols", "rw", "interpret"))
def _wass_loss(x, y, xw, yw, rows, cols, rw=512, interpret=False):
    body = functools.partial(_wass_body, rw=rw)
    f = pl.pallas_call(
        body,
        out_shape=jax.ShapeDtypeStruct((1, 1), jnp.float32),
        scratch_shapes=[
            pltpu.VMEM((2 * rows, cols), jnp.float32),
            pltpu.VMEM((2 * rows, cols), jnp.float32),
        ],
        interpret=interpret,
    )
    return f(
        x.reshape(rows, cols),
        y.reshape(rows, cols),
        xw.reshape(rows, cols),
        yw.reshape(rows, cols),
    )[0, 0]


def kernel(x, y, x_weights, y_weights, pre_sorted):
    del pre_sorted
    n = x.shape[0]
    cols = 128
    rows = n // cols
    return _wass_loss(x, y, x_weights, y_weights, rows, cols, rw=128)


# speedup vs baseline: 1.1667x; 1.0226x over previous
import functools

import jax
import jax.numpy as jnp
from jax import lax
from jax.experimental import pallas as pl
from jax.experimental.pallas import tpu as pltpu


def _cmpex_pair(av, bv, aw, bw, s, base_bit_scalar, lane_desc):
    d = av - bv
    if lane_desc is not None:
        ex = (lane_desc * d) > 0
    elif base_bit_scalar is not None:
        ex = (base_bit_scalar * d) > 0
    else:
        ex = d > 0
    return (jnp.where(ex, bv, av), jnp.where(ex, av, bv),
            jnp.where(ex, bw, aw), jnp.where(ex, aw, bw))


def _wass_body(x_ref, y_ref, xw_ref, yw_ref, out_ref, vbuf, wbuf, *, rw):
    RT = vbuf.shape[0]
    C = vbuf.shape[1]
    HRT = RT // 2
    rb = RT.bit_length() - 1
    logm = (RT * C).bit_length() - 1
    rwb = rw.bit_length() - 1
    nw = RT // rw
    f32 = jnp.float32
    i32 = jnp.int32

    def fill_tot(b, acc):
        ax, ay = acc
        ax = ax + jnp.sum(xw_ref[pl.ds(b * rw, rw), :])
        ay = ay + jnp.sum(yw_ref[pl.ds(b * rw, rw), :])
        vbuf[pl.ds(b * rw, rw), :] = x_ref[pl.ds(b * rw, rw), :]
        vbuf[pl.ds(HRT + b * rw, rw), :] = y_ref[pl.ds(b * rw, rw), :]
        return ax, ay

    wx_tot, wy_tot = lax.fori_loop(0, nw // 2, fill_tot,
                                   (jnp.float32(0.0), jnp.float32(0.0)))
    inv_x = 1.0 / wx_tot
    inv_y = -1.0 / wy_tot

    def fill_w(b, _):
        wbuf[pl.ds(b * rw, rw), :] = xw_ref[pl.ds(b * rw, rw), :] * inv_x
        wbuf[pl.ds(HRT + b * rw, rw), :] = yw_ref[pl.ds(b * rw, rw), :] * inv_y
        return 0

    lax.fori_loop(0, nw // 2, fill_w, 0)

    riota = lax.broadcasted_iota(i32, (rw, C), 0)
    ciota = lax.broadcasted_iota(i32, (rw, C), 1)

    def desc_mult(s, base):
        if s < rb:
            dsc = (base >> s) & 1
            return (1 - 2 * dsc).astype(f32)
        return (1 - 2 * ((ciota >> (s - rb)) & 1)).astype(f32)

    def substage_inwin(vv, ww, t, s, base):
        j = 1 << t
        if j >= 8:
            nseg = rw // (2 * j)
            segs_v, segs_w = [], []
            for g in range(nseg):
                lo = g * 2 * j
                av, bv = vv[lo:lo + j, :], vv[lo + j:lo + 2 * j, :]
                aw, bw = ww[lo:lo + j, :], ww[lo + j:lo + 2 * j, :]
                if s < rwb:
                    if ((lo >> s) & 1) == 0:
                        na, nb, nwa, nwb = _cmpex_pair(av, bv, aw, bw, s, None, None)
                    else:
                        nb, na, nwb, nwa = _cmpex_pair(bv, av, bw, aw, s, None, None)
                elif s < rb:
                    dscf = desc_mult(s, base)
                    na, nb, nwa, nwb = _cmpex_pair(av, bv, aw, bw, s, dscf, None)
                else:
                    lmask = desc_mult(s, 0)[:j, :]
                    na, nb, nwa, nwb = _cmpex_pair(av, bv, aw, bw, s, None, lmask)
                segs_v += [na, nb]
                segs_w += [nwa, nwb]
            return (jnp.concatenate(segs_v, axis=0),
                    jnp.concatenate(segs_w, axis=0))
        hb = (riota >> t) & 1
        if s < rwb:
            db = (riota >> s) & 1
            sgn = ((1 - 2 * hb) * (1 - 2 * db)).astype(f32)
        elif s < rb:
            sgn = (1 - 2 * hb).astype(f32) * desc_mult(s, base)
        else:
            sgn = (1 - 2 * hb).astype(f32) * desc_mult(s, 0)
        hbf = hb.astype(f32)
        pv = hbf * pltpu.roll(vv, j, 0) + (1.0 - hbf) * pltpu.roll(vv, rw - j, 0)
        pw = hbf * pltpu.roll(ww, j, 0) + (1.0 - hbf) * pltpu.roll(ww, rw - j, 0)
        ex = sgn * (vv - pv) > 0
        return jnp.where(ex, pv, vv), jnp.where(ex, pw, ww)

    def window_pass(stages):
        def body(wi, _):
            base = wi * rw
            vv = vbuf[pl.ds(base, rw), :]
            ww = wbuf[pl.ds(base, rw), :]
            for (t, s) in stages:
                vv, ww = substage_inwin(vv, ww, t, s, base)
            vbuf[pl.ds(base, rw), :] = vv
            wbuf[pl.ds(base, rw), :] = ww
            return 0
        lax.fori_loop(0, nw, body, 0)

    def pair_pass(t, s):
        jw = 1 << (t - rwb)

        def body(i, _):
            b = (i // jw) * 2 * jw + (i % jw)
            lo = b * rw
            hi = lo + jw * rw
            av = vbuf[pl.ds(lo, rw), :]
            aw = wbuf[pl.ds(lo, rw), :]
            bv = vbuf[pl.ds(hi, rw), :]
            bw = wbuf[pl.ds(hi, rw), :]
            if s < rb:
                dscf = desc_mult(s, lo)
                na, nb, nwa, nwb = _cmpex_pair(av, bv, aw, bw, s, dscf, None)
            else:
                na, nb, nwa, nwb = _cmpex_pair(av, bv, aw, bw, s, None,
                                               desc_mult(s, 0))
            vbuf[pl.ds(lo, rw), :] = na
            wbuf[pl.ds(lo, rw), :] = nwa
            vbuf[pl.ds(hi, rw), :] = nb
            wbuf[pl.ds(hi, rw), :] = nwb
            return 0

        lax.fori_loop(0, nw // 2, body, 0)

    def pair2_pass(t, s):
        k1 = t - 1 - rwb
        j1 = 1 << k1

        def body(i, _):
            b = ((i >> k1) << (k1 + 2)) | (i & (j1 - 1))
            offs = [b * rw, (b + j1) * rw, (b + 2 * j1) * rw, (b + 3 * j1) * rw]
            vs = [vbuf[pl.ds(o, rw), :] for o in offs]
            ws = [wbuf[pl.ds(o, rw), :] for o in offs]
            if s < rb:
                m_sc, m_ln = desc_mult(s, offs[0]), None
            else:
                m_sc, m_ln = None, desc_mult(s, 0)
            for (a, c) in ((0, 2), (1, 3), (0, 1), (2, 3)):
                vs[a], vs[c], ws[a], ws[c] = _cmpex_pair(
                    vs[a], vs[c], ws[a], ws[c], s, m_sc, m_ln)
            for k, o in enumerate(offs):
                vbuf[pl.ds(o, rw), :] = vs[k]
                wbuf[pl.ds(o, rw), :] = ws[k]
            return 0

        lax.fori_loop(0, nw // 4, body, 0)

    def lane_substage(vv, ww, t, s):
        jl = 1 << (t - rb)
        hb = (ciota >> (t - rb)) & 1
        db = (ciota >> (s - rb)) & 1
        sgn = ((1 - 2 * hb) * (1 - 2 * db)).astype(f32)
        hbf = hb.astype(f32)
        pv = hbf * pltpu.roll(vv, jl, 1) + (1.0 - hbf) * pltpu.roll(vv, C - jl, 1)
        pw = hbf * pltpu.roll(ww, jl, 1) + (1.0 - hbf) * pltpu.roll(ww, C - jl, 1)
        ex = sgn * (vv - pv) > 0
        return jnp.where(ex, pv, vv), jnp.where(ex, pw, ww)

    def lane_run_pass(s, tlist):
        def body(wi, _):
            base = wi * rw
            vv = vbuf[pl.ds(base, rw), :]
            ww = wbuf[pl.ds(base, rw), :]
            for t in tlist:
                vv, ww = lane_substage(vv, ww, t, s)
            vbuf[pl.ds(base, rw), :] = vv
            wbuf[pl.ds(base, rw), :] = ww
            return 0

        lax.fori_loop(0, nw, body, 0)

    def pair_window_pass(s):
        def body(i, _):
            lo = (2 * i) * rw
            hi = lo + rw
            av = vbuf[pl.ds(lo, rw), :]
            aw = wbuf[pl.ds(lo, rw), :]
            bv = vbuf[pl.ds(hi, rw), :]
            bw = wbuf[pl.ds(hi, rw), :]
            if s < rb:
                na, nb, nwa, nwb = _cmpex_pair(av, bv, aw, bw, s,
                                               desc_mult(s, lo), None)
            else:
                na, nb, nwa, nwb = _cmpex_pair(av, bv, aw, bw, s, None,
                                               desc_mult(s, 0))
            for t in range(rwb - 1, -1, -1):
                na, nwa = substage_inwin(na, nwa, t, s, lo)
                nb, nwb = substage_inwin(nb, nwb, t, s, hi)
            vbuf[pl.ds(lo, rw), :] = na
            wbuf[pl.ds(lo, rw), :] = nwa
            vbuf[pl.ds(hi, rw), :] = nb
            wbuf[pl.ds(hi, rw), :] = nwb
            return 0

        lax.fori_loop(0, nw // 2, body, 0)

    group_a = [(t, s) for s in range(1, rwb + 1) for t in range(s - 1, -1, -1)]
    window_pass(group_a)
    for s in range(rwb + 1, logm + 1):
        lane_ts = list(range(s - 1, rb - 1, -1)) if s - 1 >= rb else []
        if lane_ts:
            lane_run_pass(s, lane_ts)
        ts = list(range(min(s - 1, rb - 1), rwb, -1))
        idx = 0
        while idx < len(ts):
            if idx + 1 < len(ts) and ts[idx] - 1 > rwb:
                pair2_pass(ts[idx], s)
                idx += 2
            else:
                pair_pass(ts[idx], s)
                idx += 1
        pair_window_pass(s)

    def local_scan(ww):
        n = 1
        while n < rw:
            ww = ww + jnp.concatenate(
                [jnp.zeros((n, C), f32), ww[: rw - n, :]], axis=0)
            n *= 2
        return ww

    def pass1(b, carry):
        ww = wbuf[pl.ds(b * rw, rw), :]
        incl = local_scan(ww)
        return carry + incl[rw - 1: rw, :]

    col_tot = lax.fori_loop(0, nw, pass1, jnp.zeros((1, C), f32))

    lane_incl = col_tot
    n = 1
    while n < C:
        lane_incl = lane_incl + jnp.concatenate(
            [jnp.zeros((1, n), f32), lane_incl[:, : C - n]], axis=1)
        n *= 2
    col_off = lane_incl - col_tot

    def pass2(b, carry):
        off, loss = carry
        ww = wbuf[pl.ds(b * rw, rw), :]
        vv = vbuf[pl.ds(b * rw, rw), :]
        incl_local = local_scan(ww)
        s_incl = incl_local + off
        s_excl = s_incl - ww
        loss = loss + jnp.sum((jnp.abs(s_excl) - jnp.abs(s_incl)) * vv)
        return off + incl_local[rw - 1: rw, :], loss

    _, loss = lax.fori_loop(0, nw, pass2, (col_off, jnp.float32(0.0)))
    out_ref[...] = loss.reshape(1, 1)


@functools.partial(jax.jit, static_argnames=("rows", "cols", "rw", "interpret"))
def _wass_loss(x, y, xw, yw, rows, cols, rw=512, interpret=False):
    body = functools.partial(_wass_body, rw=rw)
    f = pl.pallas_call(
        body,
        out_shape=jax.ShapeDtypeStruct((1, 1), jnp.float32),
        scratch_shapes=[
            pltpu.VMEM((2 * rows, cols), jnp.float32),
            pltpu.VMEM((2 * rows, cols), jnp.float32),
        ],
        interpret=interpret,
    )
    return f(
        x.reshape(rows, cols),
        y.reshape(rows, cols),
        xw.reshape(rows, cols),
        yw.reshape(rows, cols),
    )[0, 0]


def kernel(x, y, x_weights, y_weights, pre_sorted):
    del pre_sorted
    n = x.shape[0]
    cols = 128
    rows = n // cols
    return _wass_loss(x, y, x_weights, y_weights, rows, cols, rw=128)


# speedup vs baseline: 1.1684x; 1.0015x over previous
import functools

import jax
import jax.numpy as jnp
from jax import lax
from jax.experimental import pallas as pl
from jax.experimental.pallas import tpu as pltpu


def _cmpex_pair(av, bv, aw, bw, s, base_bit_scalar, lane_desc):
    d = av - bv
    if lane_desc is not None:
        ex = (lane_desc * d) > 0
    elif base_bit_scalar is not None:
        ex = (base_bit_scalar * d) > 0
    else:
        ex = d > 0
    return (jnp.where(ex, bv, av), jnp.where(ex, av, bv),
            jnp.where(ex, bw, aw), jnp.where(ex, aw, bw))


def _wass_body(x_ref, y_ref, xw_ref, yw_ref, out_ref, vbuf, wbuf, *, rw):
    RT = vbuf.shape[0]
    C = vbuf.shape[1]
    HRT = RT // 2
    rb = RT.bit_length() - 1
    logm = (RT * C).bit_length() - 1
    rwb = rw.bit_length() - 1
    nw = RT // rw
    f32 = jnp.float32
    i32 = jnp.int32

    def fill_tot(b, acc):
        ax, ay = acc
        ax = ax + jnp.sum(xw_ref[pl.ds(b * rw, rw), :])
        ay = ay + jnp.sum(yw_ref[pl.ds(b * rw, rw), :])
        vbuf[pl.ds(b * rw, rw), :] = x_ref[pl.ds(b * rw, rw), :]
        vbuf[pl.ds(HRT + b * rw, rw), :] = y_ref[pl.ds(b * rw, rw), :]
        return ax, ay

    wx_tot, wy_tot = lax.fori_loop(0, nw // 2, fill_tot,
                                   (jnp.float32(0.0), jnp.float32(0.0)))
    inv_x = 1.0 / wx_tot
    inv_y = -1.0 / wy_tot

    def fill_w(b, _):
        wbuf[pl.ds(b * rw, rw), :] = xw_ref[pl.ds(b * rw, rw), :] * inv_x
        wbuf[pl.ds(HRT + b * rw, rw), :] = yw_ref[pl.ds(b * rw, rw), :] * inv_y
        return 0

    lax.fori_loop(0, nw // 2, fill_w, 0)

    riota = lax.broadcasted_iota(i32, (rw, C), 0)
    ciota = lax.broadcasted_iota(i32, (rw, C), 1)

    def desc_mult(s, base):
        if s < rb:
            dsc = (base >> s) & 1
            return (1 - 2 * dsc).astype(f32)
        return (1 - 2 * ((ciota >> (s - rb)) & 1)).astype(f32)

    def substage_inwin(vv, ww, t, s, base):
        j = 1 << t
        if j >= 8:
            nseg = rw // (2 * j)
            segs_v, segs_w = [], []
            for g in range(nseg):
                lo = g * 2 * j
                av, bv = vv[lo:lo + j, :], vv[lo + j:lo + 2 * j, :]
                aw, bw = ww[lo:lo + j, :], ww[lo + j:lo + 2 * j, :]
                if s < rwb:
                    if ((lo >> s) & 1) == 0:
                        na, nb, nwa, nwb = _cmpex_pair(av, bv, aw, bw, s, None, None)
                    else:
                        nb, na, nwb, nwa = _cmpex_pair(bv, av, bw, aw, s, None, None)
                elif s < rb:
                    dscf = desc_mult(s, base)
                    na, nb, nwa, nwb = _cmpex_pair(av, bv, aw, bw, s, dscf, None)
                else:
                    lmask = desc_mult(s, 0)[:j, :]
                    na, nb, nwa, nwb = _cmpex_pair(av, bv, aw, bw, s, None, lmask)
                segs_v += [na, nb]
                segs_w += [nwa, nwb]
            return (jnp.concatenate(segs_v, axis=0),
                    jnp.concatenate(segs_w, axis=0))
        hb = (riota >> t) & 1
        if s < rwb:
            db = (riota >> s) & 1
            sgn = ((1 - 2 * hb) * (1 - 2 * db)).astype(f32)
        elif s < rb:
            sgn = (1 - 2 * hb).astype(f32) * desc_mult(s, base)
        else:
            sgn = (1 - 2 * hb).astype(f32) * desc_mult(s, 0)
        hbf = hb.astype(f32)
        pv = hbf * pltpu.roll(vv, j, 0) + (1.0 - hbf) * pltpu.roll(vv, rw - j, 0)
        pw = hbf * pltpu.roll(ww, j, 0) + (1.0 - hbf) * pltpu.roll(ww, rw - j, 0)
        ex = sgn * (vv - pv) > 0
        return jnp.where(ex, pv, vv), jnp.where(ex, pw, ww)

    def window_pass(stages):
        def body(wi, _):
            base = wi * rw
            vv = vbuf[pl.ds(base, rw), :]
            ww = wbuf[pl.ds(base, rw), :]
            for (t, s) in stages:
                vv, ww = substage_inwin(vv, ww, t, s, base)
            vbuf[pl.ds(base, rw), :] = vv
            wbuf[pl.ds(base, rw), :] = ww
            return 0
        lax.fori_loop(0, nw, body, 0)

    def pair_pass(t, s):
        jw = 1 << (t - rwb)

        def body(i, _):
            b = (i // jw) * 2 * jw + (i % jw)
            lo = b * rw
            hi = lo + jw * rw
            av = vbuf[pl.ds(lo, rw), :]
            aw = wbuf[pl.ds(lo, rw), :]
            bv = vbuf[pl.ds(hi, rw), :]
            bw = wbuf[pl.ds(hi, rw), :]
            if s < rb:
                dscf = desc_mult(s, lo)
                na, nb, nwa, nwb = _cmpex_pair(av, bv, aw, bw, s, dscf, None)
            else:
                na, nb, nwa, nwb = _cmpex_pair(av, bv, aw, bw, s, None,
                                               desc_mult(s, 0))
            vbuf[pl.ds(lo, rw), :] = na
            wbuf[pl.ds(lo, rw), :] = nwa
            vbuf[pl.ds(hi, rw), :] = nb
            wbuf[pl.ds(hi, rw), :] = nwb
            return 0

        lax.fori_loop(0, nw // 2, body, 0)

    def pair2_pass(t, s):
        k1 = t - 1 - rwb
        j1 = 1 << k1

        def body(i, _):
            b = ((i >> k1) << (k1 + 2)) | (i & (j1 - 1))
            offs = [b * rw, (b + j1) * rw, (b + 2 * j1) * rw, (b + 3 * j1) * rw]
            vs = [vbuf[pl.ds(o, rw), :] for o in offs]
            ws = [wbuf[pl.ds(o, rw), :] for o in offs]
            if s < rb:
                m_sc, m_ln = desc_mult(s, offs[0]), None
            else:
                m_sc, m_ln = None, desc_mult(s, 0)
            for (a, c) in ((0, 2), (1, 3), (0, 1), (2, 3)):
                vs[a], vs[c], ws[a], ws[c] = _cmpex_pair(
                    vs[a], vs[c], ws[a], ws[c], s, m_sc, m_ln)
            for k, o in enumerate(offs):
                vbuf[pl.ds(o, rw), :] = vs[k]
                wbuf[pl.ds(o, rw), :] = ws[k]
            return 0

        lax.fori_loop(0, nw // 4, body, 0)

    def lane_substage(vv, ww, t, s):
        jl = 1 << (t - rb)
        hb = (ciota >> (t - rb)) & 1
        db = (ciota >> (s - rb)) & 1
        sgn = ((1 - 2 * hb) * (1 - 2 * db)).astype(f32)
        hbf = hb.astype(f32)
        pv = hbf * pltpu.roll(vv, jl, 1) + (1.0 - hbf) * pltpu.roll(vv, C - jl, 1)
        pw = hbf * pltpu.roll(ww, jl, 1) + (1.0 - hbf) * pltpu.roll(ww, C - jl, 1)
        ex = sgn * (vv - pv) > 0
        return jnp.where(ex, pv, vv), jnp.where(ex, pw, ww)

    def lane_run_pass(s, tlist):
        def body(wi, _):
            base = wi * rw
            vv = vbuf[pl.ds(base, rw), :]
            ww = wbuf[pl.ds(base, rw), :]
            for t in tlist:
                vv, ww = lane_substage(vv, ww, t, s)
            vbuf[pl.ds(base, rw), :] = vv
            wbuf[pl.ds(base, rw), :] = ww
            return 0

        lax.fori_loop(0, nw, body, 0)

    def pair_window_pass(s):
        def body(i, _):
            lo = (2 * i) * rw
            hi = lo + rw
            av = vbuf[pl.ds(lo, rw), :]
            aw = wbuf[pl.ds(lo, rw), :]
            bv = vbuf[pl.ds(hi, rw), :]
            bw = wbuf[pl.ds(hi, rw), :]
            if s < rb:
                na, nb, nwa, nwb = _cmpex_pair(av, bv, aw, bw, s,
                                               desc_mult(s, lo), None)
            else:
                na, nb, nwa, nwb = _cmpex_pair(av, bv, aw, bw, s, None,
                                               desc_mult(s, 0))
            for t in range(rwb - 1, -1, -1):
                na, nwa = substage_inwin(na, nwa, t, s, lo)
                nb, nwb = substage_inwin(nb, nwb, t, s, hi)
            vbuf[pl.ds(lo, rw), :] = na
            wbuf[pl.ds(lo, rw), :] = nwa
            vbuf[pl.ds(hi, rw), :] = nb
            wbuf[pl.ds(hi, rw), :] = nwb
            return 0

        lax.fori_loop(0, nw // 2, body, 0)

    group_a = [(t, s) for s in range(1, rwb + 1) for t in range(s - 1, -1, -1)]
    window_pass(group_a)
    for s in range(rwb + 1, logm + 1):
        lane_ts = list(range(s - 1, rb - 1, -1)) if s - 1 >= rb else []
        if lane_ts:
            lane_run_pass(s, lane_ts)
        ts = list(range(min(s - 1, rb - 1), rwb, -1))
        idx = 0
        while idx < len(ts):
            if idx + 1 < len(ts) and ts[idx] - 1 > rwb:
                pair2_pass(ts[idx], s)
                idx += 2
            else:
                pair_pass(ts[idx], s)
                idx += 1
        pair_window_pass(s)

    def local_scan(ww):
        n = 1
        while n < rw:
            ww = ww + jnp.concatenate(
                [jnp.zeros((n, C), f32), ww[: rw - n, :]], axis=0)
            n *= 2
        return ww

    def pass1(b, carry):
        ww = wbuf[pl.ds(b * rw, rw), :]
        incl = local_scan(ww)
        return carry + incl[rw - 1: rw, :]

    col_tot = lax.fori_loop(0, nw, pass1, jnp.zeros((1, C), f32))

    lane_incl = col_tot
    n = 1
    while n < C:
        lane_incl = lane_incl + jnp.concatenate(
            [jnp.zeros((1, n), f32), lane_incl[:, : C - n]], axis=1)
        n *= 2
    col_off = lane_incl - col_tot

    def pass2(b, carry):
        off, loss = carry
        ww = wbuf[pl.ds(b * rw, rw), :]
        vv = vbuf[pl.ds(b * rw, rw), :]
        incl_local = local_scan(ww)
        s_incl = incl_local + off
        s_excl = s_incl - ww
        loss = loss + jnp.sum((jnp.abs(s_excl) - jnp.abs(s_incl)) * vv)
        return off + incl_local[rw - 1: rw, :], loss

    _, loss = lax.fori_loop(0, nw, pass2, (col_off, jnp.float32(0.0)))
    out_ref[...] = loss.reshape(1, 1)


@functools.partial(jax.jit, static_argnames=("rows", "cols", "rw"))
def _wass_loss(x, y, xw, yw, rows, cols, rw=256):
    body = functools.partial(_wass_body, rw=rw)
    f = pl.pallas_call(
        body,
        out_shape=jax.ShapeDtypeStruct((1, 1), jnp.float32),
        scratch_shapes=[
            pltpu.VMEM((2 * rows, cols), jnp.float32),
            pltpu.VMEM((2 * rows, cols), jnp.float32),
        ],
    )
    return f(
        x.reshape(rows, cols),
        y.reshape(rows, cols),
        xw.reshape(rows, cols),
        yw.reshape(rows, cols),
    )[0, 0]


def kernel(x, y, x_weights, y_weights, pre_sorted):
    del pre_sorted
    n = x.shape[0]
    cols = 128
    rows = n // cols
    return _wass_loss(x, y, x_weights, y_weights, rows, cols, rw=128)
